# baseline (speedup 1.0000x reference)
import jax
import jax.numpy as jnp
from jax.experimental import pallas as pl

_BATCH_BLOCK = 16


def _add_row_kernel(x_ref, t_ref, o_ref):
    o_ref[...] = x_ref[...] + t_ref[...]


def kernel(encoded_patches, position_table):
    batch, num_patches, dim = encoded_patches.shape
    flat = num_patches * dim
    x2 = encoded_patches.reshape(batch, flat)
    t2 = position_table.reshape(1, flat)
    out = pl.pallas_call(
        _add_row_kernel,
        grid=(batch // _BATCH_BLOCK,),
        in_specs=[
            pl.BlockSpec((_BATCH_BLOCK, flat), lambda i: (i, 0)),
            pl.BlockSpec((1, flat), lambda i: (0, 0)),
        ],
        out_specs=pl.BlockSpec((_BATCH_BLOCK, flat), lambda i: (i, 0)),
        out_shape=jax.ShapeDtypeStruct((batch, flat), encoded_patches.dtype),
    )(x2, t2)
    return out.reshape(batch, num_patches, dim)


# speedup vs baseline: 1.3195x; 1.3195x over previous
import jax
import jax.numpy as jnp
from jax.experimental import pallas as pl

_BATCH_BLOCK = 16


def _add_table_kernel(x_ref, t_ref, o_ref):
    o_ref[...] = x_ref[...] + t_ref[...][None, :, :]


def kernel(encoded_patches, position_table):
    batch, num_patches, dim = encoded_patches.shape
    grid = (batch // _BATCH_BLOCK,)
    return pl.pallas_call(
        _add_table_kernel,
        grid=grid,
        in_specs=[
            pl.BlockSpec((_BATCH_BLOCK, num_patches, dim), lambda i: (i, 0, 0)),
            pl.BlockSpec((num_patches, dim), lambda i: (0, 0)),
        ],
        out_specs=pl.BlockSpec((_BATCH_BLOCK, num_patches, dim), lambda i: (i, 0, 0)),
        out_shape=jax.ShapeDtypeStruct(encoded_patches.shape, encoded_patches.dtype),
    )(encoded_patches, position_table)
